# baseline (speedup 1.0000x reference)
import jax
import jax.numpy as jnp
from jax.experimental import pallas as pl
from jax.experimental.pallas import tpu as pltpu


def _conv3x3_kernel(x_ref, ml_ref, mr_ref, w_ref, b_ref, o_ref, pk_ref):
    NB = x_ref.shape[0]
    C = x_ref.shape[1]
    H = x_ref.shape[2]
    W = x_ref.shape[3]
    S = H * W
    Cout = o_ref.shape[1]

    for nb in range(NB):
        zpad = jnp.zeros((3 * C, W), jnp.bfloat16)
        pk_ref[:, 0:W] = zpad
        pk_ref[:, W + S:] = zpad
        pk_ref[C:2 * C, W:W + S] = (
            x_ref[nb].astype(jnp.bfloat16).reshape(C, S))

        cen = pk_ref[C:2 * C, :]
        z1 = jnp.zeros((C, 1), jnp.bfloat16)
        pk_ref[0:C, :] = (jnp.concatenate([z1, cen[:, :-1]], axis=1)
                          * ml_ref[...])
        pk_ref[2 * C:3 * C, :] = (jnp.concatenate([cen[:, 1:], z1], axis=1)
                                  * mr_ref[...])

        acc = jnp.broadcast_to(b_ref[...], (Cout, S)).astype(jnp.float32)
        for kh in range(3):
            acc = acc + jnp.dot(w_ref[kh], pk_ref[:, kh * W:kh * W + S],
                                preferred_element_type=jnp.float32)

        o_ref[nb] = jnp.maximum(acc, 0.0).reshape(Cout, H, W)


def kernel(x_nchw, w_hwio, bias):
    N, Cin, H, W = x_nchw.shape
    kH, kW, _, Cout = w_hwio.shape
    assert (kH, kW) == (3, 3)

    S = H * W
    XW = S + 2 * W

    lane = jnp.arange(XW, dtype=jnp.int32) % W
    ml = (lane != 0).astype(jnp.bfloat16).reshape(1, XW)
    mr = (lane != W - 1).astype(jnp.bfloat16).reshape(1, XW)

    w_k = jnp.transpose(w_hwio, (0, 3, 1, 2)).reshape(kH, Cout, kW * Cin)
    w_k = w_k.astype(jnp.bfloat16)
    b2 = bias.reshape(Cout, 1).astype(jnp.float32)

    nb = 2 if N % 2 == 0 else 1

    out = pl.pallas_call(
        _conv3x3_kernel,
        out_shape=jax.ShapeDtypeStruct((N, Cout, H, W), jnp.float32),
        grid_spec=pltpu.PrefetchScalarGridSpec(
            num_scalar_prefetch=0,
            grid=(N // nb,),
            in_specs=[
                pl.BlockSpec((nb, Cin, H, W), lambda n: (n, 0, 0, 0)),
                pl.BlockSpec((1, XW), lambda n: (0, 0)),
                pl.BlockSpec((1, XW), lambda n: (0, 0)),
                pl.BlockSpec((kH, Cout, kW * Cin), lambda n: (0, 0, 0)),
                pl.BlockSpec((Cout, 1), lambda n: (0, 0)),
            ],
            out_specs=pl.BlockSpec((nb, Cout, H, W), lambda n: (n, 0, 0, 0)),
            scratch_shapes=[pltpu.VMEM((kW * Cin, XW), jnp.bfloat16)],
        ),
        compiler_params=pltpu.CompilerParams(
            dimension_semantics=("parallel",)),
    )(x_nchw, ml, mr, w_k, b2)

    return out


# speedup vs baseline: 1.0855x; 1.0855x over previous
import jax
import jax.numpy as jnp
from jax.experimental import pallas as pl
from jax.experimental.pallas import tpu as pltpu


def _conv3x3_kernel(x_ref, ml_ref, mr_ref, w_ref, b_ref, o_ref, pk_ref):
    C = x_ref.shape[1]
    H = x_ref.shape[2]
    W = x_ref.shape[3]
    S = H * W
    Cout = o_ref.shape[1]

    zpad = jnp.zeros((3 * C, W), jnp.bfloat16)
    pk_ref[:, 0:W] = zpad
    pk_ref[:, W + S:] = zpad
    pk_ref[C:2 * C, W:W + S] = x_ref[0].astype(jnp.bfloat16).reshape(C, S)

    cen = pk_ref[C:2 * C, :]
    z1 = jnp.zeros((C, 1), jnp.bfloat16)
    pk_ref[0:C, :] = jnp.concatenate([z1, cen[:, :-1]], axis=1) * ml_ref[...]
    pk_ref[2 * C:3 * C, :] = (jnp.concatenate([cen[:, 1:], z1], axis=1)
                              * mr_ref[...])

    acc = jnp.broadcast_to(b_ref[...], (Cout, S)).astype(jnp.float32)
    for kh in range(3):
        acc = acc + jnp.dot(w_ref[kh], pk_ref[:, kh * W:kh * W + S],
                            preferred_element_type=jnp.float32)

    o_ref[0] = jnp.maximum(acc, 0.0).reshape(Cout, H, W)


def kernel(x_nchw, w_hwio, bias):
    N, Cin, H, W = x_nchw.shape
    kH, kW, _, Cout = w_hwio.shape
    assert (kH, kW) == (3, 3)

    S = H * W
    XW = S + 2 * W

    lane = jnp.arange(XW, dtype=jnp.int32) % W
    ml = (lane != 0).astype(jnp.bfloat16).reshape(1, XW)
    mr = (lane != W - 1).astype(jnp.bfloat16).reshape(1, XW)

    w_k = jnp.transpose(w_hwio, (0, 3, 1, 2)).reshape(kH, Cout, kW * Cin)
    w_k = w_k.astype(jnp.bfloat16)
    b2 = bias.reshape(Cout, 1).astype(jnp.float32)

    out = pl.pallas_call(
        _conv3x3_kernel,
        out_shape=jax.ShapeDtypeStruct((N, Cout, H, W), jnp.float32),
        grid_spec=pltpu.PrefetchScalarGridSpec(
            num_scalar_prefetch=0,
            grid=(N,),
            in_specs=[
                pl.BlockSpec((1, Cin, H, W), lambda n: (n, 0, 0, 0)),
                pl.BlockSpec((1, XW), lambda n: (0, 0)),
                pl.BlockSpec((1, XW), lambda n: (0, 0)),
                pl.BlockSpec((kH, Cout, kW * Cin), lambda n: (0, 0, 0)),
                pl.BlockSpec((Cout, 1), lambda n: (0, 0)),
            ],
            out_specs=pl.BlockSpec((1, Cout, H, W), lambda n: (n, 0, 0, 0)),
            scratch_shapes=[pltpu.VMEM((kW * Cin, XW), jnp.bfloat16)],
        ),
        compiler_params=pltpu.CompilerParams(
            dimension_semantics=("parallel",)),
    )(x_nchw, ml, mr, w_k, b2)

    return out
